# baseline (speedup 1.0000x reference)
import functools

import jax
import jax.numpy as jnp
from jax import lax
from jax.experimental import pallas as pl
from jax.experimental.pallas import tpu as pltpu
from jax.experimental.pallas import tpu_sc as plsc

N = 50000
E = 800000
D = 128
G = 64
CH = 16
NCH = 8
NSUB = 16
RW = 2000
SBW = RW
EPS = E // NSUB
NBL = EPS // RW
NB = NBL
NACC = 50048
ZR = NACC // NSUB
ZR_LAST = N - (NSUB - 1) * ZR
BN = 1000
GRID = N // BN


def _seg_chunk(src1, dst1, xflat, out, acc, sbuf, sbuf4, dbuf, rows,
               isem, gsem, ssem, s, zrow, ch):
    pltpu.sync_copy(zrow, acc.at[pl.ds(s * ZR, ZR)])
    plsc.subcore_barrier()
    base0 = s * EPS

    def load_idx(i, p):
        e0 = base0 + i * RW
        a = pltpu.async_copy(src1.at[pl.ds(e0, RW)],
                             sbuf.at[p, pl.ds(0, RW)], isem)
        b = pltpu.async_copy(dst1.at[pl.ds(e0, RW)], dbuf.at[p], isem)
        return a, b

    def drain_idx(p):
        pltpu.make_async_copy(src1.at[pl.ds(0, RW)],
                              sbuf.at[p, pl.ds(0, RW)], isem).wait()
        pltpu.make_async_copy(dst1.at[pl.ds(0, RW)], dbuf.at[p], isem).wait()

    def fire_gather(p):
        def bias_body(t, carry):
            o = pl.multiple_of(t * 16, 16)
            v = sbuf[p, pl.ds(o, 16)]
            sbuf4[p, pl.ds(o, 16)] = v * NCH + ch
            return carry

        lax.fori_loop(0, SBW // 16, bias_body, 0)
        pltpu.async_copy(xflat.at[sbuf4.at[p, pl.ds(0, RW)]],
                         rows.at[p], gsem)

    def drain_gather(p):
        pltpu.make_async_copy(xflat.at[pl.ds(0, RW)], rows.at[p], gsem).wait()

    def fire_scatter(p):
        pltpu.async_copy(rows.at[p], acc.at[dbuf.at[p]], ssem, add=True)

    def drain_scatter(p):
        pltpu.make_async_copy(xflat.at[pl.ds(0, RW)], rows.at[p], ssem).wait()

    def step(i, p, next_gather, next_idx):
        q = 1 - p
        drain_gather(p)
        fire_scatter(p)
        if next_gather:
            drain_idx(q)
            fire_gather(q)
        drain_scatter(p)
        if next_idx:
            load_idx(i + 2, p)

    a, b = load_idx(0, 0)
    a.wait()
    b.wait()
    fire_gather(0)
    load_idx(1, 1)

    def body(t, carry):
        i0 = 2 * t
        step(i0, 0, True, True)
        step(i0 + 1, 1, True, True)
        return carry

    if NBL % 2 == 0:
        lax.fori_loop(0, (NBL - 2) // 2, body, 0)
        step(NBL - 2, 0, True, False)
        step(NBL - 1, 1, False, False)
    else:
        lax.fori_loop(0, (NBL - 3) // 2, body, 0)
        step(NBL - 3, 0, True, True)
        step(NBL - 2, 1, True, False)
        step(NBL - 1, 0, False, False)
    plsc.subcore_barrier()

    @pl.when(s < NSUB - 1)
    def _():
        pltpu.sync_copy(acc.at[pl.ds(s * ZR, ZR)],
                        out.at[pl.ds(s * ZR, ZR), pl.ds(CH * ch, CH)])

    @pl.when(s == NSUB - 1)
    def _():
        pltpu.sync_copy(acc.at[pl.ds((NSUB - 1) * ZR, ZR_LAST)],
                        out.at[pl.ds((NSUB - 1) * ZR, ZR_LAST),
                               pl.ds(CH * ch, CH)])


def _make_segsum():
    mesh = plsc.VectorSubcoreMesh(core_axis_name="c", subcore_axis_name="s")

    @functools.partial(
        pl.kernel,
        out_type=[jax.ShapeDtypeStruct((N, D), jnp.float32),
                  jax.ShapeDtypeStruct((N, D), jnp.float32)],
        mesh=mesh,
        compiler_params=pltpu.CompilerParams(use_tc_tiling_on_sc=False),
        scratch_types=[
            pltpu.VMEM_SHARED((NACC, CH), jnp.float32),
            pltpu.VMEM((2, SBW), jnp.int32),
            pltpu.VMEM((2, SBW), jnp.int32),
            pltpu.VMEM((2, RW), jnp.int32),
            pltpu.VMEM((2, RW, CH), jnp.float32),
            pltpu.SemaphoreType.DMA,
            pltpu.SemaphoreType.DMA,
            pltpu.SemaphoreType.DMA,
        ],
    )
    def seg(src_v, src_s, dst_v, dst_s, xf_v, xf_s, zrow, out_v, out_s,
            acc, sbuf, sbuf4, dbuf, rows, isem, gsem, ssem):
        c = lax.axis_index("c")
        s = lax.axis_index("s")
        for cc in (0, 1):
            @pl.when(c == cc)
            def _():
                src1 = src_v if cc == 0 else src_s
                dst1 = dst_v if cc == 0 else dst_s
                xflat = xf_v if cc == 0 else xf_s
                out = out_v if cc == 0 else out_s
                for ch in range(NCH):
                    _seg_chunk(src1, dst1, xflat, out, acc, sbuf, sbuf4,
                               dbuf, rows, isem, gsem, ssem, s, zrow, ch)

    return seg


def _make_counts():
    mesh = plsc.VectorSubcoreMesh(core_axis_name="c", subcore_axis_name="s")

    @functools.partial(
        pl.kernel,
        out_type=[jax.ShapeDtypeStruct((N, 16), jnp.float32),
                  jax.ShapeDtypeStruct((N, 16), jnp.float32)],
        mesh=mesh,
        compiler_params=pltpu.CompilerParams(use_tc_tiling_on_sc=False),
        scratch_types=[
            pltpu.VMEM_SHARED((NACC, 16), jnp.float32),
            pltpu.VMEM((2, RW), jnp.int32),
            pltpu.VMEM((RW, 16), jnp.float32),
            pltpu.SemaphoreType.DMA,
        ],
    )
    def cnt(dv1, ds1, ones_h, zcnt, out_v, out_s, acc, dbuf, ones_v, csem):
        c = lax.axis_index("c")
        s = lax.axis_index("s")
        pltpu.sync_copy(ones_h, ones_v)
        pltpu.sync_copy(zcnt, acc.at[pl.ds(s * ZR, ZR)])
        plsc.subcore_barrier()
        for cc in (0, 1):
            @pl.when(c == cc)
            def _():
                dref = (dv1, ds1)[cc]
                out = (out_v, out_s)[cc]

                def body(b, carry):
                    e0 = s * EPS + b * RW
                    pltpu.sync_copy(dref.at[pl.ds(e0, RW)], dbuf.at[0])
                    pltpu.async_copy(ones_v, acc.at[dbuf.at[0]],
                                     csem, add=True).wait()
                    return carry

                lax.fori_loop(0, NB, body, 0)
                plsc.subcore_barrier()

                @pl.when(s < NSUB - 1)
                def _w():
                    pltpu.sync_copy(acc.at[pl.ds(s * ZR, ZR)],
                                    out.at[pl.ds(s * ZR, ZR)])

                @pl.when(s == NSUB - 1)
                def _w2():
                    pltpu.sync_copy(acc.at[pl.ds((NSUB - 1) * ZR, ZR_LAST)],
                                    out.at[pl.ds((NSUB - 1) * ZR, ZR_LAST)])

    return cnt


_segsum = _make_segsum()
_counts = _make_counts()


def _layer0_body(sref, xref, cnt, wl, wr, bl, href):
    inv = 1.0 / jnp.maximum(cnt[:, 0:1], 1.0)
    href[...] = jnp.maximum(
        jnp.dot(sref[...] * inv, wl[...], preferred_element_type=jnp.float32)
        + bl[0:1, :]
        + jnp.dot(xref[...], wr[...], preferred_element_type=jnp.float32),
        0.0)


def _layer1_body(sref, xref, cnt, wl, wr, bl, batch, pool, pcnt):
    i = pl.program_id(0)
    inv = 1.0 / jnp.maximum(cnt[:, 0:1], 1.0)
    h = jnp.maximum(
        jnp.dot(sref[...] * inv, wl[...], preferred_element_type=jnp.float32)
        + bl[0:1, :]
        + jnp.dot(xref[...], wr[...], preferred_element_type=jnp.float32),
        0.0)
    oneh = (batch[0, 0, :][:, None]
            == lax.broadcasted_iota(jnp.int32, (BN, G), 1)).astype(jnp.float32)
    contrib = lax.dot_general(oneh, h, (((0,), (0,)), ((), ())),
                              preferred_element_type=jnp.float32)
    ccontrib = jnp.broadcast_to(jnp.sum(oneh, axis=0)[:, None], (G, D))

    @pl.when(i == 0)
    def _():
        pool[...] = contrib
        pcnt[...] = ccontrib

    @pl.when(i > 0)
    def _():
        pool[...] += contrib
        pcnt[...] += ccontrib


def _head_body(pv, cv, ps, cs, w, b, o):
    a = pv[...] / jnp.maximum(cv[...], 1.0)
    bb = ps[...] / jnp.maximum(cs[...], 1.0)
    rep = jnp.concatenate([a, bb], axis=1)
    o[...] = jnp.dot(rep, w[...], preferred_element_type=jnp.float32) + b[0:1, :]


def _tc_layer0(s, x, cnt, wl, wr, bl2):
    return pl.pallas_call(
        _layer0_body,
        grid=(GRID,),
        in_specs=[
            pl.BlockSpec((BN, D), lambda i: (i, 0)),
            pl.BlockSpec((BN, D), lambda i: (i, 0)),
            pl.BlockSpec((BN, 16), lambda i: (i, 0)),
            pl.BlockSpec((D, D), lambda i: (0, 0)),
            pl.BlockSpec((D, D), lambda i: (0, 0)),
            pl.BlockSpec((8, D), lambda i: (0, 0)),
        ],
        out_specs=pl.BlockSpec((BN, D), lambda i: (i, 0)),
        out_shape=jax.ShapeDtypeStruct((N, D), jnp.float32),
    )(s, x, cnt, wl, wr, bl2)


def _tc_layer1(s, x, cnt, wl, wr, bl2, batch3):
    return pl.pallas_call(
        _layer1_body,
        grid=(GRID,),
        in_specs=[
            pl.BlockSpec((BN, D), lambda i: (i, 0)),
            pl.BlockSpec((BN, D), lambda i: (i, 0)),
            pl.BlockSpec((BN, 16), lambda i: (i, 0)),
            pl.BlockSpec((D, D), lambda i: (0, 0)),
            pl.BlockSpec((D, D), lambda i: (0, 0)),
            pl.BlockSpec((8, D), lambda i: (0, 0)),
            pl.BlockSpec((1, 1, BN), lambda i: (i, 0, 0)),
        ],
        out_specs=[
            pl.BlockSpec((G, D), lambda i: (0, 0)),
            pl.BlockSpec((G, D), lambda i: (0, 0)),
        ],
        out_shape=[
            jax.ShapeDtypeStruct((G, D), jnp.float32),
            jax.ShapeDtypeStruct((G, D), jnp.float32),
        ],
    )(s, x, cnt, wl, wr, bl2, batch3)


def kernel(x_void, x_vessel, edge_index_void, edge_index_vessel, batch_void, batch_vessel,
           Wl0_void, bl0_void, Wr0_void, Wl0_vessel, bl0_vessel, Wr0_vessel,
           Wl1_void, bl1_void, Wr1_void, Wl1_vessel, bl1_vessel, Wr1_vessel,
           lin_W, lin_b):
    src_v, dst_v = edge_index_void[0], edge_index_void[1]
    src_s, dst_s = edge_index_vessel[0], edge_index_vessel[1]

    zrow = jnp.zeros((ZR, CH), jnp.float32)
    zcnt = jnp.zeros((ZR, 16), jnp.float32)
    ones_h = jnp.ones((RW, 16), jnp.float32)
    b3_v = batch_void.reshape(GRID, 1, BN)
    b3_s = batch_vessel.reshape(GRID, 1, BN)

    cnt_v, cnt_s = _counts(dst_v, dst_s, ones_h, zcnt)

    s0_v, s0_s = _segsum(src_v, src_s, dst_v, dst_s,
                         x_void.reshape(NCH * N, CH),
                         x_vessel.reshape(NCH * N, CH), zrow)
    h0_v = _tc_layer0(s0_v, x_void, cnt_v, Wl0_void, Wr0_void,
                      jnp.tile(bl0_void[None, :], (8, 1)))
    h0_s = _tc_layer0(s0_s, x_vessel, cnt_s, Wl0_vessel, Wr0_vessel,
                      jnp.tile(bl0_vessel[None, :], (8, 1)))

    s1_v, s1_s = _segsum(src_v, src_s, dst_v, dst_s,
                         h0_v.reshape(NCH * N, CH),
                         h0_s.reshape(NCH * N, CH), zrow)
    pool_v, pcnt_v = _tc_layer1(s1_v, h0_v, cnt_v, Wl1_void, Wr1_void,
                                jnp.tile(bl1_void[None, :], (8, 1)), b3_v)
    pool_s, pcnt_s = _tc_layer1(s1_s, h0_s, cnt_s, Wl1_vessel, Wr1_vessel,
                                jnp.tile(bl1_vessel[None, :], (8, 1)), b3_s)

    out = pl.pallas_call(
        _head_body,
        out_shape=jax.ShapeDtypeStruct((G, lin_W.shape[1]), jnp.float32),
    )(pool_v, pcnt_v, pool_s, pcnt_s, lin_W,
      jnp.tile(lin_b[None, :], (8, 1)))
    return out


# speedup vs baseline: 1.1996x; 1.1996x over previous
import functools

import jax
import jax.numpy as jnp
from jax import lax
from jax.experimental import pallas as pl
from jax.experimental.pallas import tpu as pltpu
from jax.experimental.pallas import tpu_sc as plsc

N = 50000
E = 800000
D = 128
G = 64
CH = 32
NCH = 4
NSUB = 16
RW = 400
SBW = RW
EPS = E // NSUB
NBL = EPS // RW
NB = NBL
NACC = 50048
ZR = NACC // NSUB
ZR_LAST = N - (NSUB - 1) * ZR
BN = 1000
GRID = N // BN


def _seg_chunk(src1, dst1, xflat, out, acc, sbuf, sbuf4, dbuf, rows,
               isem, gsem, ssem, s, zrow, ch):
    pltpu.sync_copy(zrow, acc.at[pl.ds(s * ZR, ZR)])
    plsc.subcore_barrier()
    base0 = s * EPS

    def load_idx(i, p):
        e0 = base0 + i * RW
        a = pltpu.async_copy(src1.at[pl.ds(e0, RW)],
                             sbuf.at[p, pl.ds(0, RW)], isem)
        b = pltpu.async_copy(dst1.at[pl.ds(e0, RW)], dbuf.at[p], isem)
        return a, b

    def drain_idx(p):
        pltpu.make_async_copy(src1.at[pl.ds(0, RW)],
                              sbuf.at[p, pl.ds(0, RW)], isem).wait()
        pltpu.make_async_copy(dst1.at[pl.ds(0, RW)], dbuf.at[p], isem).wait()

    def fire_gather(p):
        for t in range(SBW // 16):
            v = sbuf[p, pl.ds(16 * t, 16)]
            sbuf4[p, pl.ds(16 * t, 16)] = v * 4 + ch
        pltpu.async_copy(xflat.at[sbuf4.at[p, pl.ds(0, RW)]],
                         rows.at[p], gsem)

    def drain_gather(p):
        pltpu.make_async_copy(xflat.at[pl.ds(0, RW)], rows.at[p], gsem).wait()

    def fire_scatter(p):
        pltpu.async_copy(rows.at[p], acc.at[dbuf.at[p]], ssem, add=True)

    def drain_scatter(p):
        pltpu.make_async_copy(xflat.at[pl.ds(0, RW)], rows.at[p], ssem).wait()

    def step(i, p, next_gather, next_idx):
        q = 1 - p
        drain_gather(p)
        fire_scatter(p)
        if next_gather:
            drain_idx(q)
            fire_gather(q)
        drain_scatter(p)
        if next_idx:
            load_idx(i + 2, p)

    a, b = load_idx(0, 0)
    a.wait()
    b.wait()
    fire_gather(0)
    load_idx(1, 1)

    def body(t, carry):
        i0 = 2 * t
        step(i0, 0, True, True)
        step(i0 + 1, 1, True, True)
        return carry

    if NBL % 2 == 0:
        lax.fori_loop(0, (NBL - 2) // 2, body, 0)
        step(NBL - 2, 0, True, False)
        step(NBL - 1, 1, False, False)
    else:
        lax.fori_loop(0, (NBL - 3) // 2, body, 0)
        step(NBL - 3, 0, True, True)
        step(NBL - 2, 1, True, False)
        step(NBL - 1, 0, False, False)
    plsc.subcore_barrier()

    @pl.when(s < NSUB - 1)
    def _():
        pltpu.sync_copy(acc.at[pl.ds(s * ZR, ZR)],
                        out.at[pl.ds(s * ZR, ZR), pl.ds(CH * ch, CH)])

    @pl.when(s == NSUB - 1)
    def _():
        pltpu.sync_copy(acc.at[pl.ds((NSUB - 1) * ZR, ZR_LAST)],
                        out.at[pl.ds((NSUB - 1) * ZR, ZR_LAST),
                               pl.ds(CH * ch, CH)])


def _make_segsum():
    mesh = plsc.VectorSubcoreMesh(core_axis_name="c", subcore_axis_name="s")

    @functools.partial(
        pl.kernel,
        out_type=[jax.ShapeDtypeStruct((N, D), jnp.float32),
                  jax.ShapeDtypeStruct((N, D), jnp.float32)],
        mesh=mesh,
        compiler_params=pltpu.CompilerParams(use_tc_tiling_on_sc=False),
        scratch_types=[
            pltpu.VMEM_SHARED((NACC, CH), jnp.float32),
            pltpu.VMEM((2, SBW), jnp.int32),
            pltpu.VMEM((2, SBW), jnp.int32),
            pltpu.VMEM((2, RW), jnp.int32),
            pltpu.VMEM((2, RW, CH), jnp.float32),
            pltpu.SemaphoreType.DMA,
            pltpu.SemaphoreType.DMA,
            pltpu.SemaphoreType.DMA,
        ],
    )
    def seg(src_v, src_s, dst_v, dst_s, xf_v, xf_s, zrow, out_v, out_s,
            acc, sbuf, sbuf4, dbuf, rows, isem, gsem, ssem):
        c = lax.axis_index("c")
        s = lax.axis_index("s")
        for cc in (0, 1):
            @pl.when(c == cc)
            def _():
                src1 = src_v if cc == 0 else src_s
                dst1 = dst_v if cc == 0 else dst_s
                xflat = xf_v if cc == 0 else xf_s
                out = out_v if cc == 0 else out_s
                for ch in range(NCH):
                    _seg_chunk(src1, dst1, xflat, out, acc, sbuf, sbuf4,
                               dbuf, rows, isem, gsem, ssem, s, zrow, ch)

    return seg


def _make_counts():
    mesh = plsc.VectorSubcoreMesh(core_axis_name="c", subcore_axis_name="s")

    @functools.partial(
        pl.kernel,
        out_type=[jax.ShapeDtypeStruct((N, 16), jnp.float32),
                  jax.ShapeDtypeStruct((N, 16), jnp.float32)],
        mesh=mesh,
        compiler_params=pltpu.CompilerParams(use_tc_tiling_on_sc=False),
        scratch_types=[
            pltpu.VMEM_SHARED((NACC, 16), jnp.float32),
            pltpu.VMEM((2, RW), jnp.int32),
            pltpu.VMEM((RW, 16), jnp.float32),
            pltpu.SemaphoreType.DMA,
            pltpu.SemaphoreType.DMA,
        ],
    )
    def cnt(dv1, ds1, ones_h, zcnt, out_v, out_s, acc, dbuf, ones_v,
            csem, cisem):
        c = lax.axis_index("c")
        s = lax.axis_index("s")
        pltpu.sync_copy(ones_h, ones_v)
        pltpu.sync_copy(zcnt, acc.at[pl.ds(s * ZR, ZR)])
        plsc.subcore_barrier()
        for cc in (0, 1):
            @pl.when(c == cc)
            def _():
                dref = (dv1, ds1)[cc]
                out = (out_v, out_s)[cc]

                def load_idx(b, p):
                    return pltpu.async_copy(
                        dref.at[pl.ds(s * EPS + b * RW, RW)],
                        dbuf.at[p], cisem)

                def drain_idx(p):
                    pltpu.make_async_copy(dref.at[pl.ds(0, RW)],
                                          dbuf.at[p], cisem).wait()

                def fire_scatter(p):
                    pltpu.async_copy(ones_v, acc.at[dbuf.at[p]],
                                     csem, add=True)

                def drain_scatter(p):
                    pltpu.make_async_copy(dref.at[pl.ds(0, RW)],
                                          ones_v, csem).wait()

                def cstep(b, p, first, next_idx):
                    drain_idx(p)
                    if not first:
                        drain_scatter(1 - p)
                    fire_scatter(p)
                    if next_idx:
                        load_idx(b + 1, 1 - p)

                load_idx(0, 0)
                cstep(0, 0, True, True)

                def body(t, carry):
                    b0 = 2 * t + 1
                    cstep(b0, 1, False, True)
                    cstep(b0 + 1, 0, False, True)
                    return carry

                lax.fori_loop(0, (NB - 3) // 2, body, 0)
                cstep(NB - 2, 1, False, True)
                cstep(NB - 1, 0, False, False)
                drain_scatter(0)
                plsc.subcore_barrier()

                @pl.when(s < NSUB - 1)
                def _w():
                    pltpu.sync_copy(acc.at[pl.ds(s * ZR, ZR)],
                                    out.at[pl.ds(s * ZR, ZR)])

                @pl.when(s == NSUB - 1)
                def _w2():
                    pltpu.sync_copy(acc.at[pl.ds((NSUB - 1) * ZR, ZR_LAST)],
                                    out.at[pl.ds((NSUB - 1) * ZR, ZR_LAST)])

    return cnt


_segsum = _make_segsum()
_counts = _make_counts()


def _layer0_body(sref, xref, cnt, wl, wr, bl, href):
    inv = 1.0 / jnp.maximum(cnt[:, 0:1], 1.0)
    href[...] = jnp.maximum(
        jnp.dot(sref[...] * inv, wl[...], preferred_element_type=jnp.float32)
        + bl[0:1, :]
        + jnp.dot(xref[...], wr[...], preferred_element_type=jnp.float32),
        0.0)


def _layer1_body(sref, xref, cnt, wl, wr, bl, batch, pool, pcnt):
    i = pl.program_id(0)
    inv = 1.0 / jnp.maximum(cnt[:, 0:1], 1.0)
    h = jnp.maximum(
        jnp.dot(sref[...] * inv, wl[...], preferred_element_type=jnp.float32)
        + bl[0:1, :]
        + jnp.dot(xref[...], wr[...], preferred_element_type=jnp.float32),
        0.0)
    oneh = (batch[0, 0, :][:, None]
            == lax.broadcasted_iota(jnp.int32, (BN, G), 1)).astype(jnp.float32)
    contrib = lax.dot_general(oneh, h, (((0,), (0,)), ((), ())),
                              preferred_element_type=jnp.float32)
    ccontrib = jnp.broadcast_to(jnp.sum(oneh, axis=0)[:, None], (G, D))

    @pl.when(i == 0)
    def _():
        pool[...] = contrib
        pcnt[...] = ccontrib

    @pl.when(i > 0)
    def _():
        pool[...] += contrib
        pcnt[...] += ccontrib


def _head_body(pv, cv, ps, cs, w, b, o):
    a = pv[...] / jnp.maximum(cv[...], 1.0)
    bb = ps[...] / jnp.maximum(cs[...], 1.0)
    rep = jnp.concatenate([a, bb], axis=1)
    o[...] = jnp.dot(rep, w[...], preferred_element_type=jnp.float32) + b[0:1, :]


def _tc_layer0(s, x, cnt, wl, wr, bl2):
    return pl.pallas_call(
        _layer0_body,
        grid=(GRID,),
        in_specs=[
            pl.BlockSpec((BN, D), lambda i: (i, 0)),
            pl.BlockSpec((BN, D), lambda i: (i, 0)),
            pl.BlockSpec((BN, 16), lambda i: (i, 0)),
            pl.BlockSpec((D, D), lambda i: (0, 0)),
            pl.BlockSpec((D, D), lambda i: (0, 0)),
            pl.BlockSpec((8, D), lambda i: (0, 0)),
        ],
        out_specs=pl.BlockSpec((BN, D), lambda i: (i, 0)),
        out_shape=jax.ShapeDtypeStruct((N, D), jnp.float32),
    )(s, x, cnt, wl, wr, bl2)


def _tc_layer1(s, x, cnt, wl, wr, bl2, batch3):
    return pl.pallas_call(
        _layer1_body,
        grid=(GRID,),
        in_specs=[
            pl.BlockSpec((BN, D), lambda i: (i, 0)),
            pl.BlockSpec((BN, D), lambda i: (i, 0)),
            pl.BlockSpec((BN, 16), lambda i: (i, 0)),
            pl.BlockSpec((D, D), lambda i: (0, 0)),
            pl.BlockSpec((D, D), lambda i: (0, 0)),
            pl.BlockSpec((8, D), lambda i: (0, 0)),
            pl.BlockSpec((1, 1, BN), lambda i: (i, 0, 0)),
        ],
        out_specs=[
            pl.BlockSpec((G, D), lambda i: (0, 0)),
            pl.BlockSpec((G, D), lambda i: (0, 0)),
        ],
        out_shape=[
            jax.ShapeDtypeStruct((G, D), jnp.float32),
            jax.ShapeDtypeStruct((G, D), jnp.float32),
        ],
    )(s, x, cnt, wl, wr, bl2, batch3)


def kernel(x_void, x_vessel, edge_index_void, edge_index_vessel, batch_void, batch_vessel,
           Wl0_void, bl0_void, Wr0_void, Wl0_vessel, bl0_vessel, Wr0_vessel,
           Wl1_void, bl1_void, Wr1_void, Wl1_vessel, bl1_vessel, Wr1_vessel,
           lin_W, lin_b):
    src_v, dst_v = edge_index_void[0], edge_index_void[1]
    src_s, dst_s = edge_index_vessel[0], edge_index_vessel[1]

    zrow = jnp.zeros((ZR, CH), jnp.float32)
    zcnt = jnp.zeros((ZR, 16), jnp.float32)
    ones_h = jnp.ones((RW, 16), jnp.float32)
    b3_v = batch_void.reshape(GRID, 1, BN)
    b3_s = batch_vessel.reshape(GRID, 1, BN)

    cnt_v, cnt_s = _counts(dst_v, dst_s, ones_h, zcnt)

    s0_v, s0_s = _segsum(src_v, src_s, dst_v, dst_s,
                         x_void.reshape(NCH * N, CH),
                         x_vessel.reshape(NCH * N, CH), zrow)
    h0_v = _tc_layer0(s0_v, x_void, cnt_v, Wl0_void, Wr0_void,
                      jnp.tile(bl0_void[None, :], (8, 1)))
    h0_s = _tc_layer0(s0_s, x_vessel, cnt_s, Wl0_vessel, Wr0_vessel,
                      jnp.tile(bl0_vessel[None, :], (8, 1)))

    s1_v, s1_s = _segsum(src_v, src_s, dst_v, dst_s,
                         h0_v.reshape(NCH * N, CH),
                         h0_s.reshape(NCH * N, CH), zrow)
    pool_v, pcnt_v = _tc_layer1(s1_v, h0_v, cnt_v, Wl1_void, Wr1_void,
                                jnp.tile(bl1_void[None, :], (8, 1)), b3_v)
    pool_s, pcnt_s = _tc_layer1(s1_s, h0_s, cnt_s, Wl1_vessel, Wr1_vessel,
                                jnp.tile(bl1_vessel[None, :], (8, 1)), b3_s)

    out = pl.pallas_call(
        _head_body,
        out_shape=jax.ShapeDtypeStruct((G, lin_W.shape[1]), jnp.float32),
    )(pool_v, pcnt_v, pool_s, pcnt_s, lin_W,
      jnp.tile(lin_b[None, :], (8, 1)))
    return out
